# baseline (speedup 1.0000x reference)
import numpy as np
import jax
import jax.numpy as jnp
from jax.experimental import pallas as pl
from jax.experimental.pallas import tpu as pltpu

_D_MODEL = 2048
_N_EXPERT = 16
_N_TOKENS = 16384
_BM = 4096
_BD = 512
_GM = _N_TOKENS // _BM
_GD = _D_MODEL // _BD

_EPS = np.asarray(
    jax.random.normal(jax.random.key(42), (_N_TOKENS, _N_EXPERT), dtype=jnp.float32)
)

_IOTA16 = np.arange(_N_EXPERT, dtype=np.float32).reshape(1, _N_EXPERT)


def _router_body(x_ref, wc_ref, bc_ref, eps_ref, iota_ref, out_ref, ids_ref, acc_ref):
    d = pl.program_id(1)
    part = jnp.dot(x_ref[...], wc_ref[...], preferred_element_type=jnp.float32)

    @pl.when(d == 0)
    def _():
        acc_ref[...] = part + bc_ref[...]

    @pl.when(d != 0)
    def _():
        acc_ref[...] += part

    @pl.when(d == _GD - 1)
    def _():
        acc = acc_ref[...]
        logits = acc[:, :_N_EXPERT]
        nlog = acc[:, _N_EXPERT:]
        sp = jnp.maximum(nlog, 0.0) + jnp.log1p(jnp.exp(-jnp.abs(nlog)))
        noisy = logits + eps_ref[...] * sp

        iota = iota_ref[...]
        m1 = jnp.max(noisy, axis=1, keepdims=True)
        id1 = jnp.min(jnp.where(noisy == m1, iota, 16.0), axis=1, keepdims=True)
        sel1 = iota == id1
        masked = jnp.where(sel1, -jnp.inf, noisy)
        m2 = jnp.max(masked, axis=1, keepdims=True)
        id2 = jnp.min(jnp.where(masked == m2, iota, 16.0), axis=1, keepdims=True)
        sel2 = iota == id2

        p2 = jnp.exp(m2 - m1)
        inv_z = 1.0 / (1.0 + p2)
        out_ref[...] = jnp.where(sel1, inv_z, 0.0) + jnp.where(sel2, p2 * inv_z, 0.0)
        ids_ref[...] = jnp.concatenate([id1, id2], axis=1).astype(jnp.int32)


def kernel(x, W, b, Wn, bn):
    wc = jnp.concatenate([W, Wn], axis=0).T
    bc = jnp.concatenate([b, bn]).reshape(1, 2 * _N_EXPERT)
    eps = jnp.asarray(_EPS)
    out, ids = pl.pallas_call(
        _router_body,
        grid=(_GM, _GD),
        in_specs=[
            pl.BlockSpec((_BM, _BD), lambda m, d: (m, d)),
            pl.BlockSpec((_BD, 2 * _N_EXPERT), lambda m, d: (d, 0)),
            pl.BlockSpec((1, 2 * _N_EXPERT), lambda m, d: (0, 0)),
            pl.BlockSpec((_BM, _N_EXPERT), lambda m, d: (m, 0)),
            pl.BlockSpec((1, _N_EXPERT), lambda m, d: (0, 0)),
        ],
        out_specs=[
            pl.BlockSpec((_BM, _N_EXPERT), lambda m, d: (m, 0)),
            pl.BlockSpec((_BM, 2), lambda m, d: (m, 0)),
        ],
        out_shape=[
            jax.ShapeDtypeStruct((_N_TOKENS, _N_EXPERT), jnp.float32),
            jax.ShapeDtypeStruct((_N_TOKENS, 2), jnp.int32),
        ],
        scratch_shapes=[pltpu.VMEM((_BM, 2 * _N_EXPERT), jnp.float32)],
        compiler_params=pltpu.CompilerParams(
            dimension_semantics=("parallel", "arbitrary"),
        ),
    )(x, wc, bc, eps, jnp.asarray(_IOTA16))
    return (out, ids)


# speedup vs baseline: 1.0107x; 1.0107x over previous
import numpy as np
import jax
import jax.numpy as jnp
from jax.experimental import pallas as pl
from jax.experimental.pallas import tpu as pltpu

_D_MODEL = 2048
_N_EXPERT = 16
_N_TOKENS = 16384
_BM = 2048
_BD = 1024
_GM = _N_TOKENS // _BM
_GD = _D_MODEL // _BD

_EPS = np.asarray(
    jax.random.normal(jax.random.key(42), (_N_TOKENS, _N_EXPERT), dtype=jnp.float32)
)

_IOTA16 = np.arange(_N_EXPERT, dtype=np.float32).reshape(1, _N_EXPERT)


def _router_body(x_ref, wc_ref, bc_ref, eps_ref, iota_ref, out_ref, ids_ref, acc_ref):
    d = pl.program_id(1)
    part = jnp.dot(x_ref[...], wc_ref[...], preferred_element_type=jnp.float32)

    @pl.when(d == 0)
    def _():
        acc_ref[...] = part + bc_ref[...]

    @pl.when(d != 0)
    def _():
        acc_ref[...] += part

    @pl.when(d == _GD - 1)
    def _():
        acc = acc_ref[...]
        logits = acc[:, :_N_EXPERT]
        nlog = acc[:, _N_EXPERT:]
        sp = jnp.maximum(nlog, 0.0) + jnp.log1p(jnp.exp(-jnp.abs(nlog)))
        noisy = logits + eps_ref[...] * sp

        iota = iota_ref[...]
        m1 = jnp.max(noisy, axis=1, keepdims=True)
        id1 = jnp.min(jnp.where(noisy == m1, iota, 16.0), axis=1, keepdims=True)
        sel1 = iota == id1
        masked = jnp.where(sel1, -jnp.inf, noisy)
        m2 = jnp.max(masked, axis=1, keepdims=True)
        id2 = jnp.min(jnp.where(masked == m2, iota, 16.0), axis=1, keepdims=True)
        sel2 = iota == id2

        p2 = jnp.exp(m2 - m1)
        inv_z = 1.0 / (1.0 + p2)
        out_ref[...] = jnp.where(sel1, inv_z, 0.0) + jnp.where(sel2, p2 * inv_z, 0.0)
        ids_ref[...] = jnp.concatenate([id1, id2], axis=1).astype(jnp.int32)


def kernel(x, W, b, Wn, bn):
    wc = jnp.concatenate([W, Wn], axis=0).T
    bc = jnp.concatenate([b, bn]).reshape(1, 2 * _N_EXPERT)
    eps = jnp.asarray(_EPS)
    out, ids = pl.pallas_call(
        _router_body,
        grid=(_GM, _GD),
        in_specs=[
            pl.BlockSpec((_BM, _BD), lambda m, d: (m, d)),
            pl.BlockSpec((_BD, 2 * _N_EXPERT), lambda m, d: (d, 0)),
            pl.BlockSpec((1, 2 * _N_EXPERT), lambda m, d: (0, 0)),
            pl.BlockSpec((_BM, _N_EXPERT), lambda m, d: (m, 0)),
            pl.BlockSpec((1, _N_EXPERT), lambda m, d: (0, 0)),
        ],
        out_specs=[
            pl.BlockSpec((_BM, _N_EXPERT), lambda m, d: (m, 0)),
            pl.BlockSpec((_BM, 2), lambda m, d: (m, 0)),
        ],
        out_shape=[
            jax.ShapeDtypeStruct((_N_TOKENS, _N_EXPERT), jnp.float32),
            jax.ShapeDtypeStruct((_N_TOKENS, 2), jnp.int32),
        ],
        scratch_shapes=[pltpu.VMEM((_BM, 2 * _N_EXPERT), jnp.float32)],
        compiler_params=pltpu.CompilerParams(
            dimension_semantics=("parallel", "arbitrary"),
        ),
    )(x, wc, bc, eps, jnp.asarray(_IOTA16))
    return (out, ids)


# speedup vs baseline: 1.2352x; 1.2220x over previous
import numpy as np
import jax
import jax.numpy as jnp
from jax.experimental import pallas as pl
from jax.experimental.pallas import tpu as pltpu

_D_MODEL = 2048
_N_EXPERT = 16
_N_TOKENS = 16384
_BM = 2048

_EPS_T = np.ascontiguousarray(
    np.asarray(
        jax.random.normal(jax.random.key(42), (_N_TOKENS, _N_EXPERT), dtype=jnp.float32)
    ).T
)


def _router_body(x_ref, wc_ref, bc_ref, epsT_ref, out_ref, ids_ref):
    acc = jnp.dot(x_ref[...], wc_ref[...], preferred_element_type=jnp.float32)
    acc = acc + bc_ref[...]
    accT = acc.T
    logitsT = accT[:_N_EXPERT, :]
    nlogT = accT[_N_EXPERT:, :]
    spT = jnp.maximum(nlogT, 0.0) + jnp.log1p(jnp.exp(-jnp.abs(nlogT)))
    noisyT = logitsT + epsT_ref[...] * spT

    iota = jax.lax.broadcasted_iota(jnp.int32, (_N_EXPERT, _BM), 0)
    m1 = jnp.max(noisyT, axis=0, keepdims=True)
    id1 = jnp.min(jnp.where(noisyT == m1, iota, _N_EXPERT), axis=0, keepdims=True)
    sel1 = iota == id1
    masked = jnp.where(sel1, -jnp.inf, noisyT)
    m2 = jnp.max(masked, axis=0, keepdims=True)
    id2 = jnp.min(jnp.where(masked == m2, iota, _N_EXPERT), axis=0, keepdims=True)
    sel2 = iota == id2

    p2 = jnp.exp(m2 - m1)
    inv_z = 1.0 / (1.0 + p2)
    outT = jnp.where(sel1, inv_z, 0.0) + jnp.where(sel2, p2 * inv_z, 0.0)
    out_ref[...] = outT.T
    ids_ref[...] = jnp.concatenate([id1, id2], axis=0).T


def kernel(x, W, b, Wn, bn):
    wc = jnp.concatenate([W, Wn], axis=0).T
    bc = jnp.concatenate([b, bn]).reshape(1, 2 * _N_EXPERT)
    epsT = jnp.asarray(_EPS_T)
    grid = _N_TOKENS // _BM
    out, ids = pl.pallas_call(
        _router_body,
        grid=(grid,),
        in_specs=[
            pl.BlockSpec((_BM, _D_MODEL), lambda i: (i, 0)),
            pl.BlockSpec((_D_MODEL, 2 * _N_EXPERT), lambda i: (0, 0)),
            pl.BlockSpec((1, 2 * _N_EXPERT), lambda i: (0, 0)),
            pl.BlockSpec((_N_EXPERT, _BM), lambda i: (0, i)),
        ],
        out_specs=[
            pl.BlockSpec((_BM, _N_EXPERT), lambda i: (i, 0)),
            pl.BlockSpec((_BM, 2), lambda i: (i, 0)),
        ],
        out_shape=[
            jax.ShapeDtypeStruct((_N_TOKENS, _N_EXPERT), jnp.float32),
            jax.ShapeDtypeStruct((_N_TOKENS, 2), jnp.int32),
        ],
        compiler_params=pltpu.CompilerParams(
            dimension_semantics=("parallel",),
        ),
    )(x, wc, bc, epsT)
    return (out, ids)
